# baseline (speedup 1.0000x reference)
import jax
import jax.numpy as jnp
from jax import lax
from jax.experimental import pallas as pl
from jax.experimental.pallas import tpu as pltpu
from jax.experimental.pallas import tpu_sc as plsc

N = 10000
E = 320000
R = 16
H = 128
C = 16
NR = N * R

NC = 2
NS = 16
CH = 80
EB = 2000

EH = E // NS
ET = E // (NC * NS)
NP1 = 10240
RT1 = NP1 // NS
NP = 10112
RT = NP // NS


def _sc_mesh():
    return plsc.VectorSubcoreMesh(core_axis_name="c", subcore_axis_name="s")


def _idx_l1(eb1, eb2, eb3, b, gidx_v, dstc_v):
    for k in range(5):
        sv = eb1[pl.ds(b + k * 16, 16)]
        dv = eb2[pl.ds(b + k * 16, 16)]
        tv = eb3[pl.ds(b + k * 16, 16)]
        gidx_v[pl.ds(k * 16, 16)] = tv * N + sv
        dstc_v[pl.ds(k * 16, 16)] = dv


def _scale_rows(rows, wbuf, b):
    for k in range(5):
        wv = wbuf[pl.ds(b + k * 16, 16)]
        for jj in range(16):
            ws = wv[jj]
            r = k * 16 + jj
            for f in range(8):
                rows[r, pl.ds(f * 16, 16)] = rows[r, pl.ds(f * 16, 16)] * ws


def _sc1_body(src_hbm, dst_hbm, typ_hbm, w0_hbm,
              hpart_hbm, w_hbm,
              eb1, eb2, eb3, wbuf, fbuf,
              rowsA, rowsB,
              gidxA, gidxB, dstcA, dstcB,
              segblk, hseg2d, ones80,
              cnt_sh, acc_sh,
              gsem0, gsem1, wsem0, ssem, stsem, hsem):
    c = lax.axis_index("c")
    s = lax.axis_index("s")

    zero16 = jnp.zeros((16,), jnp.float32)
    one16 = jnp.ones((16,), jnp.float32)

    @pl.loop(0, EB // 16)
    def _(i):
        fbuf[pl.ds(i * 16, 16)] = zero16

    @pl.loop(0, CH)
    def _(i):
        for f in range(8):
            rowsA[i, pl.ds(f * 16, 16)] = zero16

    for k in range(5):
        ones80[pl.ds(k * 16, 16)] = one16

    for j in range(NR // NS // EB):
        pltpu.sync_copy(fbuf, cnt_sh.at[pl.ds(s * (NR // NS) + j * EB, EB)])
    for j in range(RT1 // CH):
        pltpu.sync_copy(rowsA, acc_sh.at[pl.ds(s * RT1 + j * CH, CH)])

    plsc.subcore_barrier()

    @pl.loop(0, EH // EB)
    def _(blk):
        eb = s * EH + blk * EB
        d1 = pltpu.async_copy(dst_hbm.at[pl.ds(eb, EB)], eb1, stsem)
        d2 = pltpu.async_copy(typ_hbm.at[pl.ds(eb, EB)], eb2, stsem)
        d1.wait()
        d2.wait()

        descs = []
        for j in range(EB // CH):
            for k in range(5):
                dv = eb1[pl.ds(j * CH + k * 16, 16)]
                tv = eb2[pl.ds(j * CH + k * 16, 16)]
                hseg2d[j, pl.ds(k * 16, 16)] = dv * R + tv
            descs.append(pltpu.async_copy(
                ones80, cnt_sh.at[hseg2d.at[j]], hsem, add=True))
        for d in descs:
            d.wait()

    plsc.subcore_barrier()

    for j in range(NR // NS // EB):
        base = s * (NR // NS) + j * EB
        pltpu.sync_copy(cnt_sh.at[pl.ds(base, EB)], fbuf)

        @pl.loop(0, EB // 16)
        def _(i):
            v = fbuf[pl.ds(i * 16, 16)]
            fbuf[pl.ds(i * 16, 16)] = 1.0 / jnp.maximum(v, 1.0)

        pltpu.sync_copy(fbuf, cnt_sh.at[pl.ds(base, EB)])

    plsc.subcore_barrier()

    @pl.loop(0, ET // EB)
    def _(blk):
        eb = c * (E // NC) + s * ET + blk * EB
        d1 = pltpu.async_copy(src_hbm.at[pl.ds(eb, EB)], eb1, stsem)
        d2 = pltpu.async_copy(dst_hbm.at[pl.ds(eb, EB)], eb2, stsem)
        d3 = pltpu.async_copy(typ_hbm.at[pl.ds(eb, EB)], eb3, stsem)
        d1.wait()
        d2.wait()
        d3.wait()

        @pl.loop(0, EB // 16)
        def _(i):
            dv = eb2[pl.ds(i * 16, 16)]
            tv = eb3[pl.ds(i * 16, 16)]
            segblk[pl.ds(i * 16, 16)] = dv * R + tv

        wdescs = []
        for j in range(EB // CH):
            wdescs.append(pltpu.async_copy(
                cnt_sh.at[segblk.at[pl.ds(j * CH, CH)]],
                wbuf.at[pl.ds(j * CH, CH)], wsem0))
        for d in wdescs:
            d.wait()

        @pl.loop(0, EB // CH // 2)
        def _(t):
            b0 = (2 * t) * CH
            b1 = b0 + CH
            _idx_l1(eb1, eb2, eb3, b0, gidxA, dstcA)
            dg0 = pltpu.async_copy(w0_hbm.at[gidxA], rowsA, gsem0)
            _idx_l1(eb1, eb2, eb3, b1, gidxB, dstcB)
            dg1 = pltpu.async_copy(w0_hbm.at[gidxB], rowsB, gsem1)

            dg0.wait()
            _scale_rows(rowsA, wbuf, b0)
            ds0 = pltpu.async_copy(rowsA, acc_sh.at[dstcA], ssem, add=True)

            dg1.wait()
            _scale_rows(rowsB, wbuf, b1)
            ds1 = pltpu.async_copy(rowsB, acc_sh.at[dstcB], ssem, add=True)

            ds0.wait()
            ds1.wait()

        b = (EB // CH - 1) * CH
        _idx_l1(eb1, eb2, eb3, b, gidxA, dstcA)
        dg0 = pltpu.async_copy(w0_hbm.at[gidxA], rowsA, gsem0)
        dg0.wait()
        _scale_rows(rowsA, wbuf, b)
        pltpu.sync_copy(rowsA, acc_sh.at[dstcA], add=True)

        pltpu.sync_copy(wbuf, w_hbm.at[pl.ds(eb, EB)])

    plsc.subcore_barrier()

    pltpu.sync_copy(acc_sh.at[pl.ds(s * RT1, RT1)],
                    hpart_hbm.at[c, pl.ds(s * RT1, RT1)])


def _run_sc1(src, dst, typ, w0flat):
    kern = pl.kernel(
        _sc1_body,
        out_type=[
            jax.ShapeDtypeStruct((NC, NP1, H), jnp.float32),
            jax.ShapeDtypeStruct((E,), jnp.float32),
        ],
        mesh=_sc_mesh(),
        scratch_types=[
            pltpu.VMEM((EB,), jnp.int32),
            pltpu.VMEM((EB,), jnp.int32),
            pltpu.VMEM((EB,), jnp.int32),
            pltpu.VMEM((EB,), jnp.float32),
            pltpu.VMEM((EB,), jnp.float32),
            pltpu.VMEM((CH, H), jnp.float32),
            pltpu.VMEM((CH, H), jnp.float32),
            pltpu.VMEM((CH,), jnp.int32),
            pltpu.VMEM((CH,), jnp.int32),
            pltpu.VMEM((CH,), jnp.int32),
            pltpu.VMEM((CH,), jnp.int32),
            pltpu.VMEM((EB,), jnp.int32),
            pltpu.VMEM((EB // CH, CH), jnp.int32),
            pltpu.VMEM((CH,), jnp.float32),
            pltpu.VMEM_SHARED((NR,), jnp.float32),
            pltpu.VMEM_SHARED((NP1, H), jnp.float32),
            pltpu.SemaphoreType.DMA,
            pltpu.SemaphoreType.DMA,
            pltpu.SemaphoreType.DMA,
            pltpu.SemaphoreType.DMA,
            pltpu.SemaphoreType.DMA,
            pltpu.SemaphoreType.DMA,
        ],
        name="rgcn_sc_layer1",
    )
    return kern(src, dst, typ, w0flat)


def _idx_l2(eb1, eb2, eb3, b, gidx_v, dstc_v):
    for k in range(5):
        sv = eb1[pl.ds(b + k * 16, 16)]
        dv = eb2[pl.ds(b + k * 16, 16)]
        tv = eb3[pl.ds(b + k * 16, 16)]
        gidx_v[pl.ds(k * 16, 16)] = sv * 2 + (tv >> 3)
        dstc_v[pl.ds(k * 16, 16)] = dv


def _scale_och(och, rows3, wstage, eb3, b):
    for k in range(5):
        wv = wstage[pl.ds(b + k * 16, 16)]
        tvv = eb3[pl.ds(b + k * 16, 16)]
        for jj in range(16):
            r = k * 16 + jj
            off = (tvv[jj] & 7) * C
            och[r, pl.ds(off, 16)] = rows3[r, pl.ds(off, 16)] * wv[jj]


def _clear_och(och, eb3, b):
    zero16 = jnp.zeros((16,), jnp.float32)
    for k in range(5):
        tvv = eb3[pl.ds(b + k * 16, 16)]
        for jj in range(16):
            r = k * 16 + jj
            off = (tvv[jj] & 7) * C
            och[r, pl.ds(off, 16)] = zero16


def _sc2_body(src_hbm, dst_hbm, typ_hbm, y_hbm, w_hbm,
              opart_hbm,
              eb1, eb2, eb3, wstage, rows3A, rows3B,
              ochA, ochB, gidxA, gidxB, dstcA, dstcB, oacc_sh,
              gsem0, gsem1, ssem, stsem):
    c = lax.axis_index("c")
    s = lax.axis_index("s")

    zero16 = jnp.zeros((16,), jnp.float32)

    @pl.loop(0, CH)
    def _(i):
        for f in range(H // 16):
            ochA[i, pl.ds(f * 16, 16)] = zero16
            ochB[i, pl.ds(f * 16, 16)] = zero16

    for j in range(RT // CH):
        pltpu.sync_copy(ochA, oacc_sh.at[pl.ds(s * RT + j * CH, CH)])
    pltpu.sync_copy(ochA.at[pl.ds(0, RT % CH)],
                    oacc_sh.at[pl.ds(s * RT + (RT // CH) * CH, RT % CH)])

    plsc.subcore_barrier()

    @pl.loop(0, ET // EB)
    def _(blk):
        eb = c * (E // NC) + s * ET + blk * EB
        d1 = pltpu.async_copy(src_hbm.at[pl.ds(eb, EB)], eb1, stsem)
        d2 = pltpu.async_copy(dst_hbm.at[pl.ds(eb, EB)], eb2, stsem)
        d3 = pltpu.async_copy(typ_hbm.at[pl.ds(eb, EB)], eb3, stsem)
        d4 = pltpu.async_copy(w_hbm.at[pl.ds(eb, EB)], wstage, stsem)
        d1.wait()
        d2.wait()
        d3.wait()
        d4.wait()

        @pl.loop(0, EB // CH // 2)
        def _(t):
            b0 = (2 * t) * CH
            b1 = b0 + CH
            _idx_l2(eb1, eb2, eb3, b0, gidxA, dstcA)
            dg0 = pltpu.async_copy(y_hbm.at[gidxA], rows3A, gsem0)
            _idx_l2(eb1, eb2, eb3, b1, gidxB, dstcB)
            dg1 = pltpu.async_copy(y_hbm.at[gidxB], rows3B, gsem1)

            dg0.wait()
            _scale_och(ochA, rows3A, wstage, eb3, b0)
            ds0 = pltpu.async_copy(ochA, oacc_sh.at[dstcA], ssem, add=True)
            dg1.wait()
            _scale_och(ochB, rows3B, wstage, eb3, b1)
            ds1 = pltpu.async_copy(ochB, oacc_sh.at[dstcB], ssem, add=True)
            ds0.wait()
            _clear_och(ochA, eb3, b0)
            ds1.wait()
            _clear_och(ochB, eb3, b1)

        b = (EB // CH - 1) * CH
        _idx_l2(eb1, eb2, eb3, b, gidxA, dstcA)
        dg0 = pltpu.async_copy(y_hbm.at[gidxA], rows3A, gsem0)
        dg0.wait()
        _scale_och(ochA, rows3A, wstage, eb3, b)
        pltpu.sync_copy(ochA, oacc_sh.at[dstcA], add=True)
        _clear_och(ochA, eb3, b)

    plsc.subcore_barrier()

    pltpu.sync_copy(oacc_sh.at[pl.ds(s * RT, RT)],
                    opart_hbm.at[c, pl.ds(s * RT, RT)])


def _run_sc2(src, dst, typ, yflat, w):
    kern = pl.kernel(
        _sc2_body,
        out_type=jax.ShapeDtypeStruct((NC, NP, H), jnp.float32),
        mesh=_sc_mesh(),
        scratch_types=[
            pltpu.VMEM((EB,), jnp.int32),
            pltpu.VMEM((EB,), jnp.int32),
            pltpu.VMEM((EB,), jnp.int32),
            pltpu.VMEM((EB,), jnp.float32),
            pltpu.VMEM((CH, H), jnp.float32),
            pltpu.VMEM((CH, H), jnp.float32),
            pltpu.VMEM((CH, H), jnp.float32),
            pltpu.VMEM((CH, H), jnp.float32),
            pltpu.VMEM((CH,), jnp.int32),
            pltpu.VMEM((CH,), jnp.int32),
            pltpu.VMEM((CH,), jnp.int32),
            pltpu.VMEM((CH,), jnp.int32),
            pltpu.VMEM_SHARED((NP, H), jnp.float32),
            pltpu.SemaphoreType.DMA,
            pltpu.SemaphoreType.DMA,
            pltpu.SemaphoreType.DMA,
            pltpu.SemaphoreType.DMA,
        ],
        name="rgcn_sc_layer2",
    )
    return kern(src, dst, typ, yflat, w)


def _tc1_body(hp0, hp1, root0, b0, w1t, root1, y_out, z_out):
    h = jnp.maximum(hp0[0] + hp1[0] + root0[...] + b0[...], 0.0)
    y_out[...] = jnp.dot(h, w1t[...], preferred_element_type=jnp.float32)
    z_out[...] = jnp.dot(h, root1[...], preferred_element_type=jnp.float32)


def _run_tc1(hpart, root0, b0, w1t, root1):
    BN = 1000
    grid = (N // BN,)
    return pl.pallas_call(
        _tc1_body,
        grid=grid,
        in_specs=[
            pl.BlockSpec((1, BN, H), lambda i: (0, i, 0)),
            pl.BlockSpec((1, BN, H), lambda i: (1, i, 0)),
            pl.BlockSpec((BN, H), lambda i: (i, 0)),
            pl.BlockSpec((1, H), lambda i: (0, 0)),
            pl.BlockSpec((H, R * C), lambda i: (0, 0)),
            pl.BlockSpec((H, C), lambda i: (0, 0)),
        ],
        out_specs=[
            pl.BlockSpec((BN, R * C), lambda i: (i, 0)),
            pl.BlockSpec((BN, C), lambda i: (i, 0)),
        ],
        out_shape=[
            jax.ShapeDtypeStruct((N, R * C), jnp.float32),
            jax.ShapeDtypeStruct((N, C), jnp.float32),
        ],
    )(hpart, hpart, root0, b0, w1t, root1)


def _tc2_body(o0, o1, z, b1, out):
    ow = o0[0] + o1[0]
    slog = z[...] + b1[...]
    for g in range(H // C):
        slog = slog + ow[:, g * C:(g + 1) * C]
    m = jnp.max(slog, axis=1, keepdims=True)
    ex = jnp.exp(slog - m)
    lse = jnp.log(jnp.sum(ex, axis=1, keepdims=True))
    out[...] = slog - m - lse


def _run_tc2(opart, z, b1):
    BN = 1000
    grid = (N // BN,)
    return pl.pallas_call(
        _tc2_body,
        grid=grid,
        in_specs=[
            pl.BlockSpec((1, BN, H), lambda i: (0, i, 0)),
            pl.BlockSpec((1, BN, H), lambda i: (1, i, 0)),
            pl.BlockSpec((BN, C), lambda i: (i, 0)),
            pl.BlockSpec((1, C), lambda i: (0, 0)),
        ],
        out_specs=pl.BlockSpec((BN, C), lambda i: (i, 0)),
        out_shape=jax.ShapeDtypeStruct((N, C), jnp.float32),
    )(opart, opart, z, b1)


def kernel(edge_index, edge_type, W0, root0, b0, W1, root1, b1):
    src = edge_index[0]
    dst = edge_index[1]
    typ = edge_type

    w0flat = W0.reshape(R * N, H)
    w1t = jnp.transpose(W1, (1, 0, 2)).reshape(H, R * C)

    hpart, w = _run_sc1(src, dst, typ, w0flat)
    y2, z = _run_tc1(hpart, root0, b0.reshape(1, H), w1t, root1)
    yflat = y2.reshape(N * 2, H)
    opart = _run_sc2(src, dst, typ, yflat, w)
    out = _run_tc2(opart, z, b1.reshape(1, C))
    return out


# speedup vs baseline: 1.0794x; 1.0794x over previous
import jax
import jax.numpy as jnp
from jax import lax
from jax.experimental import pallas as pl
from jax.experimental.pallas import tpu as pltpu
from jax.experimental.pallas import tpu_sc as plsc

N = 10000
E = 320000
R = 16
H = 128
C = 16
NR = N * R

NC = 2
NS = 16
CH = 80
EB = 2000

EH = E // NS
ET = E // (NC * NS)
NP1 = 10240
RT1 = NP1 // NS
NP = 10112
RT = NP // NS


def _sc_mesh():
    return plsc.VectorSubcoreMesh(core_axis_name="c", subcore_axis_name="s")


def _idx_l1(eb1, eb2, eb3, b, gidx_v, dstc_v):
    for k in range(5):
        sv = eb1[pl.ds(b + k * 16, 16)]
        dv = eb2[pl.ds(b + k * 16, 16)]
        tv = eb3[pl.ds(b + k * 16, 16)]
        gidx_v[pl.ds(k * 16, 16)] = tv * N + sv
        dstc_v[pl.ds(k * 16, 16)] = dv


def _scale_rows(rows, wbuf, b):
    for k in range(5):
        wv = wbuf[pl.ds(b + k * 16, 16)]
        for jj in range(16):
            ws = wv[jj]
            r = k * 16 + jj
            for f in range(8):
                rows[r, pl.ds(f * 16, 16)] = rows[r, pl.ds(f * 16, 16)] * ws


def _sc1_body(src_hbm, dst_hbm, typ_hbm, w0_hbm,
              hpart_hbm, w_hbm,
              eb1, eb2, eb3, wbuf, fbuf,
              rowsA, rowsB,
              gidxA, gidxB, dstcA, dstcB,
              segblk, hseg2d, ones80,
              cnt_sh, acc_sh,
              gsem0, gsem1, wsem0, ssemA, ssemB, stsem, hsem):
    c = lax.axis_index("c")
    s = lax.axis_index("s")

    zero16 = jnp.zeros((16,), jnp.float32)
    one16 = jnp.ones((16,), jnp.float32)

    @pl.loop(0, EB // 16)
    def _(i):
        fbuf[pl.ds(i * 16, 16)] = zero16

    @pl.loop(0, CH)
    def _(i):
        for f in range(8):
            rowsA[i, pl.ds(f * 16, 16)] = zero16

    for k in range(5):
        ones80[pl.ds(k * 16, 16)] = one16

    for j in range(NR // NS // EB):
        pltpu.sync_copy(fbuf, cnt_sh.at[pl.ds(s * (NR // NS) + j * EB, EB)])
    for j in range(RT1 // CH):
        pltpu.sync_copy(rowsA, acc_sh.at[pl.ds(s * RT1 + j * CH, CH)])

    plsc.subcore_barrier()

    @pl.loop(0, EH // EB)
    def _(blk):
        eb = s * EH + blk * EB
        d1 = pltpu.async_copy(dst_hbm.at[pl.ds(eb, EB)], eb1, stsem)
        d2 = pltpu.async_copy(typ_hbm.at[pl.ds(eb, EB)], eb2, stsem)
        d1.wait()
        d2.wait()

        descs = []
        for j in range(EB // CH):
            for k in range(5):
                dv = eb1[pl.ds(j * CH + k * 16, 16)]
                tv = eb2[pl.ds(j * CH + k * 16, 16)]
                hseg2d[j, pl.ds(k * 16, 16)] = dv * R + tv
            descs.append(pltpu.async_copy(
                ones80, cnt_sh.at[hseg2d.at[j]], hsem, add=True))
        for d in descs:
            d.wait()

    plsc.subcore_barrier()

    for j in range(NR // NS // EB):
        base = s * (NR // NS) + j * EB
        pltpu.sync_copy(cnt_sh.at[pl.ds(base, EB)], fbuf)

        @pl.loop(0, EB // 16)
        def _(i):
            v = fbuf[pl.ds(i * 16, 16)]
            fbuf[pl.ds(i * 16, 16)] = 1.0 / jnp.maximum(v, 1.0)

        pltpu.sync_copy(fbuf, cnt_sh.at[pl.ds(base, EB)])

    plsc.subcore_barrier()

    @pl.loop(0, ET // EB)
    def _(blk):
        eb = c * (E // NC) + s * ET + blk * EB
        d1 = pltpu.async_copy(src_hbm.at[pl.ds(eb, EB)], eb1, stsem)
        d2 = pltpu.async_copy(dst_hbm.at[pl.ds(eb, EB)], eb2, stsem)
        d3 = pltpu.async_copy(typ_hbm.at[pl.ds(eb, EB)], eb3, stsem)
        d1.wait()
        d2.wait()
        d3.wait()

        @pl.loop(0, EB // 16)
        def _(i):
            dv = eb2[pl.ds(i * 16, 16)]
            tv = eb3[pl.ds(i * 16, 16)]
            segblk[pl.ds(i * 16, 16)] = dv * R + tv

        wdescs = []
        for j in range(EB // CH):
            wdescs.append(pltpu.async_copy(
                cnt_sh.at[segblk.at[pl.ds(j * CH, CH)]],
                wbuf.at[pl.ds(j * CH, CH)], wsem0))
        for d in wdescs:
            d.wait()

        @pl.loop(0, EB // CH // 2)
        def _(t):
            b0 = (2 * t) * CH
            b1 = b0 + CH
            _idx_l1(eb1, eb2, eb3, b0, gidxA, dstcA)

            @pl.when(t > 0)
            def _():
                pltpu.make_async_copy(
                    w0_hbm.at[pl.ds(0, CH)], rowsA, ssemA).wait()

            dg0 = pltpu.async_copy(w0_hbm.at[gidxA], rowsA, gsem0)
            _idx_l1(eb1, eb2, eb3, b1, gidxB, dstcB)

            @pl.when(t > 0)
            def _():
                pltpu.make_async_copy(
                    w0_hbm.at[pl.ds(0, CH)], rowsB, ssemB).wait()

            dg1 = pltpu.async_copy(w0_hbm.at[gidxB], rowsB, gsem1)

            dg0.wait()
            _scale_rows(rowsA, wbuf, b0)
            pltpu.async_copy(rowsA, acc_sh.at[dstcA], ssemA, add=True)

            dg1.wait()
            _scale_rows(rowsB, wbuf, b1)
            pltpu.async_copy(rowsB, acc_sh.at[dstcB], ssemB, add=True)

        pltpu.make_async_copy(w0_hbm.at[pl.ds(0, CH)], rowsA, ssemA).wait()
        pltpu.make_async_copy(w0_hbm.at[pl.ds(0, CH)], rowsB, ssemB).wait()

        b = (EB // CH - 1) * CH
        _idx_l1(eb1, eb2, eb3, b, gidxA, dstcA)
        dg0 = pltpu.async_copy(w0_hbm.at[gidxA], rowsA, gsem0)
        dg0.wait()
        _scale_rows(rowsA, wbuf, b)
        pltpu.sync_copy(rowsA, acc_sh.at[dstcA], add=True)

        pltpu.sync_copy(wbuf, w_hbm.at[pl.ds(eb, EB)])

    plsc.subcore_barrier()

    pltpu.sync_copy(acc_sh.at[pl.ds(s * RT1, RT1)],
                    hpart_hbm.at[c, pl.ds(s * RT1, RT1)])


def _run_sc1(src, dst, typ, w0flat):
    kern = pl.kernel(
        _sc1_body,
        out_type=[
            jax.ShapeDtypeStruct((NC, NP1, H), jnp.float32),
            jax.ShapeDtypeStruct((E,), jnp.float32),
        ],
        mesh=_sc_mesh(),
        scratch_types=[
            pltpu.VMEM((EB,), jnp.int32),
            pltpu.VMEM((EB,), jnp.int32),
            pltpu.VMEM((EB,), jnp.int32),
            pltpu.VMEM((EB,), jnp.float32),
            pltpu.VMEM((EB,), jnp.float32),
            pltpu.VMEM((CH, H), jnp.float32),
            pltpu.VMEM((CH, H), jnp.float32),
            pltpu.VMEM((CH,), jnp.int32),
            pltpu.VMEM((CH,), jnp.int32),
            pltpu.VMEM((CH,), jnp.int32),
            pltpu.VMEM((CH,), jnp.int32),
            pltpu.VMEM((EB,), jnp.int32),
            pltpu.VMEM((EB // CH, CH), jnp.int32),
            pltpu.VMEM((CH,), jnp.float32),
            pltpu.VMEM_SHARED((NR,), jnp.float32),
            pltpu.VMEM_SHARED((NP1, H), jnp.float32),
            pltpu.SemaphoreType.DMA,
            pltpu.SemaphoreType.DMA,
            pltpu.SemaphoreType.DMA,
            pltpu.SemaphoreType.DMA,
            pltpu.SemaphoreType.DMA,
            pltpu.SemaphoreType.DMA,
            pltpu.SemaphoreType.DMA,
        ],
        name="rgcn_sc_layer1",
    )
    return kern(src, dst, typ, w0flat)


def _idx_l2(eb1, eb2, eb3, b, gidx_v, dstc_v):
    for k in range(5):
        sv = eb1[pl.ds(b + k * 16, 16)]
        dv = eb2[pl.ds(b + k * 16, 16)]
        tv = eb3[pl.ds(b + k * 16, 16)]
        gidx_v[pl.ds(k * 16, 16)] = sv * 2 + (tv >> 3)
        dstc_v[pl.ds(k * 16, 16)] = dv


def _scale_och(och, rows3, wstage, eb3, b, offv):
    for k in range(5):
        wv = wstage[pl.ds(b + k * 16, 16)]
        tvv = eb3[pl.ds(b + k * 16, 16)]
        offv[pl.ds(k * 16, 16)] = (tvv & 7) * C
        for jj in range(16):
            r = k * 16 + jj
            off = (tvv[jj] & 7) * C
            och[r, pl.ds(off, 16)] = rows3[r, pl.ds(off, 16)] * wv[jj]


def _clear_och(och, offv):
    zero16 = jnp.zeros((16,), jnp.float32)
    for k in range(5):
        ov = offv[pl.ds(k * 16, 16)]
        for jj in range(16):
            r = k * 16 + jj
            och[r, pl.ds(ov[jj], 16)] = zero16


def _sc2_body(src_hbm, dst_hbm, typ_hbm, y_hbm, w_hbm,
              opart_hbm,
              eb1, eb2, eb3, wstage, rows3A, rows3B,
              ochA, ochB, gidxA, gidxB, dstcA, dstcB, offA, offB, oacc_sh,
              gsem0, gsem1, ssemA, ssemB, stsem):
    c = lax.axis_index("c")
    s = lax.axis_index("s")

    zero16 = jnp.zeros((16,), jnp.float32)

    @pl.loop(0, CH)
    def _(i):
        for f in range(H // 16):
            ochA[i, pl.ds(f * 16, 16)] = zero16
            ochB[i, pl.ds(f * 16, 16)] = zero16

    for j in range(RT // CH):
        pltpu.sync_copy(ochA, oacc_sh.at[pl.ds(s * RT + j * CH, CH)])
    pltpu.sync_copy(ochA.at[pl.ds(0, RT % CH)],
                    oacc_sh.at[pl.ds(s * RT + (RT // CH) * CH, RT % CH)])

    plsc.subcore_barrier()

    @pl.loop(0, ET // EB)
    def _(blk):
        eb = c * (E // NC) + s * ET + blk * EB
        d1 = pltpu.async_copy(src_hbm.at[pl.ds(eb, EB)], eb1, stsem)
        d2 = pltpu.async_copy(dst_hbm.at[pl.ds(eb, EB)], eb2, stsem)
        d3 = pltpu.async_copy(typ_hbm.at[pl.ds(eb, EB)], eb3, stsem)
        d4 = pltpu.async_copy(w_hbm.at[pl.ds(eb, EB)], wstage, stsem)
        d1.wait()
        d2.wait()
        d3.wait()
        d4.wait()

        @pl.loop(0, EB // CH // 2)
        def _(t):
            b0 = (2 * t) * CH
            b1 = b0 + CH
            _idx_l2(eb1, eb2, eb3, b0, gidxA, dstcA)
            dg0 = pltpu.async_copy(y_hbm.at[gidxA], rows3A, gsem0)
            _idx_l2(eb1, eb2, eb3, b1, gidxB, dstcB)
            dg1 = pltpu.async_copy(y_hbm.at[gidxB], rows3B, gsem1)

            dg0.wait()

            @pl.when(t > 0)
            def _():
                pltpu.make_async_copy(
                    y_hbm.at[pl.ds(0, CH)], ochA, ssemA).wait()
                _clear_och(ochA, offA)

            _scale_och(ochA, rows3A, wstage, eb3, b0, offA)
            pltpu.async_copy(ochA, oacc_sh.at[dstcA], ssemA, add=True)

            dg1.wait()

            @pl.when(t > 0)
            def _():
                pltpu.make_async_copy(
                    y_hbm.at[pl.ds(0, CH)], ochB, ssemB).wait()
                _clear_och(ochB, offB)

            _scale_och(ochB, rows3B, wstage, eb3, b1, offB)
            pltpu.async_copy(ochB, oacc_sh.at[dstcB], ssemB, add=True)

        pltpu.make_async_copy(y_hbm.at[pl.ds(0, CH)], ochA, ssemA).wait()
        _clear_och(ochA, offA)
        pltpu.make_async_copy(y_hbm.at[pl.ds(0, CH)], ochB, ssemB).wait()
        _clear_och(ochB, offB)

        b = (EB // CH - 1) * CH
        _idx_l2(eb1, eb2, eb3, b, gidxA, dstcA)
        dg0 = pltpu.async_copy(y_hbm.at[gidxA], rows3A, gsem0)
        dg0.wait()
        _scale_och(ochA, rows3A, wstage, eb3, b, offA)
        pltpu.sync_copy(ochA, oacc_sh.at[dstcA], add=True)
        _clear_och(ochA, offA)

    plsc.subcore_barrier()

    pltpu.sync_copy(oacc_sh.at[pl.ds(s * RT, RT)],
                    opart_hbm.at[c, pl.ds(s * RT, RT)])


def _run_sc2(src, dst, typ, yflat, w):
    kern = pl.kernel(
        _sc2_body,
        out_type=jax.ShapeDtypeStruct((NC, NP, H), jnp.float32),
        mesh=_sc_mesh(),
        scratch_types=[
            pltpu.VMEM((EB,), jnp.int32),
            pltpu.VMEM((EB,), jnp.int32),
            pltpu.VMEM((EB,), jnp.int32),
            pltpu.VMEM((EB,), jnp.float32),
            pltpu.VMEM((CH, H), jnp.float32),
            pltpu.VMEM((CH, H), jnp.float32),
            pltpu.VMEM((CH, H), jnp.float32),
            pltpu.VMEM((CH, H), jnp.float32),
            pltpu.VMEM((CH,), jnp.int32),
            pltpu.VMEM((CH,), jnp.int32),
            pltpu.VMEM((CH,), jnp.int32),
            pltpu.VMEM((CH,), jnp.int32),
            pltpu.VMEM((CH,), jnp.int32),
            pltpu.VMEM((CH,), jnp.int32),
            pltpu.VMEM_SHARED((NP, H), jnp.float32),
            pltpu.SemaphoreType.DMA,
            pltpu.SemaphoreType.DMA,
            pltpu.SemaphoreType.DMA,
            pltpu.SemaphoreType.DMA,
            pltpu.SemaphoreType.DMA,
        ],
        name="rgcn_sc_layer2",
    )
    return kern(src, dst, typ, yflat, w)


def _tc1_body(hp0, hp1, root0, b0, w1t, root1, y_out, z_out):
    h = jnp.maximum(hp0[0] + hp1[0] + root0[...] + b0[...], 0.0)
    y_out[...] = jnp.dot(h, w1t[...], preferred_element_type=jnp.float32)
    z_out[...] = jnp.dot(h, root1[...], preferred_element_type=jnp.float32)


def _run_tc1(hpart, root0, b0, w1t, root1):
    BN = 1000
    grid = (N // BN,)
    return pl.pallas_call(
        _tc1_body,
        grid=grid,
        in_specs=[
            pl.BlockSpec((1, BN, H), lambda i: (0, i, 0)),
            pl.BlockSpec((1, BN, H), lambda i: (1, i, 0)),
            pl.BlockSpec((BN, H), lambda i: (i, 0)),
            pl.BlockSpec((1, H), lambda i: (0, 0)),
            pl.BlockSpec((H, R * C), lambda i: (0, 0)),
            pl.BlockSpec((H, C), lambda i: (0, 0)),
        ],
        out_specs=[
            pl.BlockSpec((BN, R * C), lambda i: (i, 0)),
            pl.BlockSpec((BN, C), lambda i: (i, 0)),
        ],
        out_shape=[
            jax.ShapeDtypeStruct((N, R * C), jnp.float32),
            jax.ShapeDtypeStruct((N, C), jnp.float32),
        ],
    )(hpart, hpart, root0, b0, w1t, root1)


def _tc2_body(o0, o1, z, b1, out):
    ow = o0[0] + o1[0]
    slog = z[...] + b1[...]
    for g in range(H // C):
        slog = slog + ow[:, g * C:(g + 1) * C]
    m = jnp.max(slog, axis=1, keepdims=True)
    ex = jnp.exp(slog - m)
    lse = jnp.log(jnp.sum(ex, axis=1, keepdims=True))
    out[...] = slog - m - lse


def _run_tc2(opart, z, b1):
    BN = 1000
    grid = (N // BN,)
    return pl.pallas_call(
        _tc2_body,
        grid=grid,
        in_specs=[
            pl.BlockSpec((1, BN, H), lambda i: (0, i, 0)),
            pl.BlockSpec((1, BN, H), lambda i: (1, i, 0)),
            pl.BlockSpec((BN, C), lambda i: (i, 0)),
            pl.BlockSpec((1, C), lambda i: (0, 0)),
        ],
        out_specs=pl.BlockSpec((BN, C), lambda i: (i, 0)),
        out_shape=jax.ShapeDtypeStruct((N, C), jnp.float32),
    )(opart, opart, z, b1)


def kernel(edge_index, edge_type, W0, root0, b0, W1, root1, b1):
    src = edge_index[0]
    dst = edge_index[1]
    typ = edge_type

    w0flat = W0.reshape(R * N, H)
    w1t = jnp.transpose(W1, (1, 0, 2)).reshape(H, R * C)

    hpart, w = _run_sc1(src, dst, typ, w0flat)
    y2, z = _run_tc1(hpart, root0, b0.reshape(1, H), w1t, root1)
    yflat = y2.reshape(N * 2, H)
    opart = _run_sc2(src, dst, typ, yflat, w)
    out = _run_tc2(opart, z, b1.reshape(1, C))
    return out


# speedup vs baseline: 1.0860x; 1.0061x over previous
import jax
import jax.numpy as jnp
from jax import lax
from jax.experimental import pallas as pl
from jax.experimental.pallas import tpu as pltpu
from jax.experimental.pallas import tpu_sc as plsc

N = 10000
E = 320000
R = 16
H = 128
C = 16
NR = N * R

NC = 2
NS = 16
CH = 80
EB = 2000

EH = E // NS
ET = E // (NC * NS)
NP1 = 10240
RT1 = NP1 // NS
NP = 10112
RT = NP // NS


def _sc_mesh():
    return plsc.VectorSubcoreMesh(core_axis_name="c", subcore_axis_name="s")


def _idx_l1(eb1, eb2, eb3, b, gidx_v, dstc_v):
    for k in range(5):
        sv = eb1[pl.ds(b + k * 16, 16)]
        dv = eb2[pl.ds(b + k * 16, 16)]
        tv = eb3[pl.ds(b + k * 16, 16)]
        gidx_v[pl.ds(k * 16, 16)] = tv * N + sv
        dstc_v[pl.ds(k * 16, 16)] = dv


def _scale_rows(rows, wbuf, b):
    for k in range(5):
        wv = wbuf[pl.ds(b + k * 16, 16)]
        for jj in range(16):
            ws = wv[jj]
            r = k * 16 + jj
            for f in range(8):
                rows[r, pl.ds(f * 16, 16)] = rows[r, pl.ds(f * 16, 16)] * ws


def _sc1_body(src_hbm, dst_hbm, typ_hbm, w0_hbm,
              hpart_hbm, w_hbm,
              eb1, eb2, eb3, wbuf, fbuf,
              rowsA, rowsB,
              gidxA, gidxB, dstcA, dstcB,
              segblk, hseg2d, ones80,
              cnt_sh, acc_sh,
              gsem0, gsem1, wsem0, ssemA, ssemB, stsem, hsem):
    c = lax.axis_index("c")
    s = lax.axis_index("s")

    zero16 = jnp.zeros((16,), jnp.float32)
    one16 = jnp.ones((16,), jnp.float32)

    @pl.loop(0, EB // 16)
    def _(i):
        fbuf[pl.ds(i * 16, 16)] = zero16

    @pl.loop(0, CH)
    def _(i):
        for f in range(8):
            rowsA[i, pl.ds(f * 16, 16)] = zero16

    for k in range(5):
        ones80[pl.ds(k * 16, 16)] = one16

    for j in range(NR // NS // EB):
        pltpu.sync_copy(fbuf, cnt_sh.at[pl.ds(s * (NR // NS) + j * EB, EB)])
    for j in range(RT1 // CH):
        pltpu.sync_copy(rowsA, acc_sh.at[pl.ds(s * RT1 + j * CH, CH)])

    plsc.subcore_barrier()

    @pl.loop(0, EH // EB)
    def _(blk):
        eb = s * EH + blk * EB
        d1 = pltpu.async_copy(dst_hbm.at[pl.ds(eb, EB)], eb1, stsem)
        d2 = pltpu.async_copy(typ_hbm.at[pl.ds(eb, EB)], eb2, stsem)
        d1.wait()
        d2.wait()

        descs = []
        for j in range(EB // CH):
            for k in range(5):
                dv = eb1[pl.ds(j * CH + k * 16, 16)]
                tv = eb2[pl.ds(j * CH + k * 16, 16)]
                hseg2d[j, pl.ds(k * 16, 16)] = dv * R + tv
            descs.append(pltpu.async_copy(
                ones80, cnt_sh.at[hseg2d.at[j]], hsem, add=True))
        for d in descs:
            d.wait()

    plsc.subcore_barrier()

    @pl.loop(0, ET // EB)
    def _(blk):
        eb = c * (E // NC) + s * ET + blk * EB
        d1 = pltpu.async_copy(src_hbm.at[pl.ds(eb, EB)], eb1, stsem)
        d2 = pltpu.async_copy(dst_hbm.at[pl.ds(eb, EB)], eb2, stsem)
        d3 = pltpu.async_copy(typ_hbm.at[pl.ds(eb, EB)], eb3, stsem)
        d1.wait()
        d2.wait()
        d3.wait()

        @pl.loop(0, EB // 16)
        def _(i):
            dv = eb2[pl.ds(i * 16, 16)]
            tv = eb3[pl.ds(i * 16, 16)]
            segblk[pl.ds(i * 16, 16)] = dv * R + tv

        wdescs = []
        for j in range(EB // CH):
            wdescs.append(pltpu.async_copy(
                cnt_sh.at[segblk.at[pl.ds(j * CH, CH)]],
                wbuf.at[pl.ds(j * CH, CH)], wsem0))
        for d in wdescs:
            d.wait()

        @pl.loop(0, EB // 16)
        def _(i):
            v = wbuf[pl.ds(i * 16, 16)]
            wbuf[pl.ds(i * 16, 16)] = 1.0 / jnp.maximum(v, 1.0)

        @pl.loop(0, EB // CH // 2)
        def _(t):
            b0 = (2 * t) * CH
            b1 = b0 + CH
            _idx_l1(eb1, eb2, eb3, b0, gidxA, dstcA)

            @pl.when(t > 0)
            def _():
                pltpu.make_async_copy(
                    w0_hbm.at[pl.ds(0, CH)], rowsA, ssemA).wait()

            dg0 = pltpu.async_copy(w0_hbm.at[gidxA], rowsA, gsem0)
            _idx_l1(eb1, eb2, eb3, b1, gidxB, dstcB)

            @pl.when(t > 0)
            def _():
                pltpu.make_async_copy(
                    w0_hbm.at[pl.ds(0, CH)], rowsB, ssemB).wait()

            dg1 = pltpu.async_copy(w0_hbm.at[gidxB], rowsB, gsem1)

            dg0.wait()
            _scale_rows(rowsA, wbuf, b0)
            pltpu.async_copy(rowsA, acc_sh.at[dstcA], ssemA, add=True)

            dg1.wait()
            _scale_rows(rowsB, wbuf, b1)
            pltpu.async_copy(rowsB, acc_sh.at[dstcB], ssemB, add=True)

        pltpu.make_async_copy(w0_hbm.at[pl.ds(0, CH)], rowsA, ssemA).wait()
        pltpu.make_async_copy(w0_hbm.at[pl.ds(0, CH)], rowsB, ssemB).wait()

        b = (EB // CH - 1) * CH
        _idx_l1(eb1, eb2, eb3, b, gidxA, dstcA)
        dg0 = pltpu.async_copy(w0_hbm.at[gidxA], rowsA, gsem0)
        dg0.wait()
        _scale_rows(rowsA, wbuf, b)
        pltpu.sync_copy(rowsA, acc_sh.at[dstcA], add=True)

        pltpu.sync_copy(wbuf, w_hbm.at[pl.ds(eb, EB)])

    plsc.subcore_barrier()

    pltpu.sync_copy(acc_sh.at[pl.ds(s * RT1, RT1)],
                    hpart_hbm.at[c, pl.ds(s * RT1, RT1)])


def _run_sc1(src, dst, typ, w0flat):
    kern = pl.kernel(
        _sc1_body,
        out_type=[
            jax.ShapeDtypeStruct((NC, NP1, H), jnp.float32),
            jax.ShapeDtypeStruct((E,), jnp.float32),
        ],
        mesh=_sc_mesh(),
        scratch_types=[
            pltpu.VMEM((EB,), jnp.int32),
            pltpu.VMEM((EB,), jnp.int32),
            pltpu.VMEM((EB,), jnp.int32),
            pltpu.VMEM((EB,), jnp.float32),
            pltpu.VMEM((EB,), jnp.float32),
            pltpu.VMEM((CH, H), jnp.float32),
            pltpu.VMEM((CH, H), jnp.float32),
            pltpu.VMEM((CH,), jnp.int32),
            pltpu.VMEM((CH,), jnp.int32),
            pltpu.VMEM((CH,), jnp.int32),
            pltpu.VMEM((CH,), jnp.int32),
            pltpu.VMEM((EB,), jnp.int32),
            pltpu.VMEM((EB // CH, CH), jnp.int32),
            pltpu.VMEM((CH,), jnp.float32),
            pltpu.VMEM_SHARED((NR,), jnp.float32),
            pltpu.VMEM_SHARED((NP1, H), jnp.float32),
            pltpu.SemaphoreType.DMA,
            pltpu.SemaphoreType.DMA,
            pltpu.SemaphoreType.DMA,
            pltpu.SemaphoreType.DMA,
            pltpu.SemaphoreType.DMA,
            pltpu.SemaphoreType.DMA,
            pltpu.SemaphoreType.DMA,
        ],
        name="rgcn_sc_layer1",
    )
    return kern(src, dst, typ, w0flat)


def _idx_l2(eb1, eb2, eb3, b, gidx_v, dstc_v):
    for k in range(5):
        sv = eb1[pl.ds(b + k * 16, 16)]
        dv = eb2[pl.ds(b + k * 16, 16)]
        tv = eb3[pl.ds(b + k * 16, 16)]
        gidx_v[pl.ds(k * 16, 16)] = sv * 2 + (tv >> 3)
        dstc_v[pl.ds(k * 16, 16)] = dv


def _scale_och(och, rows3, wstage, eb3, b, offv):
    for k in range(5):
        wv = wstage[pl.ds(b + k * 16, 16)]
        tvv = eb3[pl.ds(b + k * 16, 16)]
        offv[pl.ds(k * 16, 16)] = (tvv & 7) * C
        for jj in range(16):
            r = k * 16 + jj
            off = (tvv[jj] & 7) * C
            och[r, pl.ds(off, 16)] = rows3[r, pl.ds(off, 16)] * wv[jj]


def _clear_och(och, offv):
    zero16 = jnp.zeros((16,), jnp.float32)
    for k in range(5):
        ov = offv[pl.ds(k * 16, 16)]
        for jj in range(16):
            r = k * 16 + jj
            och[r, pl.ds(ov[jj], 16)] = zero16


def _sc2_body(src_hbm, dst_hbm, typ_hbm, y_hbm, w_hbm,
              opart_hbm,
              eb1, eb2, eb3, wstage, rows3A, rows3B,
              ochA, ochB, gidxA, gidxB, dstcA, dstcB, offA, offB, oacc_sh,
              gsem0, gsem1, ssemA, ssemB, stsem):
    c = lax.axis_index("c")
    s = lax.axis_index("s")

    zero16 = jnp.zeros((16,), jnp.float32)

    @pl.loop(0, CH)
    def _(i):
        for f in range(H // 16):
            ochA[i, pl.ds(f * 16, 16)] = zero16
            ochB[i, pl.ds(f * 16, 16)] = zero16

    for j in range(RT // CH):
        pltpu.sync_copy(ochA, oacc_sh.at[pl.ds(s * RT + j * CH, CH)])
    pltpu.sync_copy(ochA.at[pl.ds(0, RT % CH)],
                    oacc_sh.at[pl.ds(s * RT + (RT // CH) * CH, RT % CH)])

    plsc.subcore_barrier()

    @pl.loop(0, ET // EB)
    def _(blk):
        eb = c * (E // NC) + s * ET + blk * EB
        d1 = pltpu.async_copy(src_hbm.at[pl.ds(eb, EB)], eb1, stsem)
        d2 = pltpu.async_copy(dst_hbm.at[pl.ds(eb, EB)], eb2, stsem)
        d3 = pltpu.async_copy(typ_hbm.at[pl.ds(eb, EB)], eb3, stsem)
        d4 = pltpu.async_copy(w_hbm.at[pl.ds(eb, EB)], wstage, stsem)
        d1.wait()
        d2.wait()
        d3.wait()
        d4.wait()

        @pl.loop(0, EB // CH // 2)
        def _(t):
            b0 = (2 * t) * CH
            b1 = b0 + CH
            _idx_l2(eb1, eb2, eb3, b0, gidxA, dstcA)
            dg0 = pltpu.async_copy(y_hbm.at[gidxA], rows3A, gsem0)
            _idx_l2(eb1, eb2, eb3, b1, gidxB, dstcB)
            dg1 = pltpu.async_copy(y_hbm.at[gidxB], rows3B, gsem1)

            dg0.wait()

            @pl.when(t > 0)
            def _():
                pltpu.make_async_copy(
                    y_hbm.at[pl.ds(0, CH)], ochA, ssemA).wait()
                _clear_och(ochA, offA)

            _scale_och(ochA, rows3A, wstage, eb3, b0, offA)
            pltpu.async_copy(ochA, oacc_sh.at[dstcA], ssemA, add=True)

            dg1.wait()

            @pl.when(t > 0)
            def _():
                pltpu.make_async_copy(
                    y_hbm.at[pl.ds(0, CH)], ochB, ssemB).wait()
                _clear_och(ochB, offB)

            _scale_och(ochB, rows3B, wstage, eb3, b1, offB)
            pltpu.async_copy(ochB, oacc_sh.at[dstcB], ssemB, add=True)

        pltpu.make_async_copy(y_hbm.at[pl.ds(0, CH)], ochA, ssemA).wait()
        _clear_och(ochA, offA)
        pltpu.make_async_copy(y_hbm.at[pl.ds(0, CH)], ochB, ssemB).wait()
        _clear_och(ochB, offB)

        b = (EB // CH - 1) * CH
        _idx_l2(eb1, eb2, eb3, b, gidxA, dstcA)
        dg0 = pltpu.async_copy(y_hbm.at[gidxA], rows3A, gsem0)
        dg0.wait()
        _scale_och(ochA, rows3A, wstage, eb3, b, offA)
        pltpu.sync_copy(ochA, oacc_sh.at[dstcA], add=True)
        _clear_och(ochA, offA)

    plsc.subcore_barrier()

    pltpu.sync_copy(oacc_sh.at[pl.ds(s * RT, RT)],
                    opart_hbm.at[c, pl.ds(s * RT, RT)])


def _run_sc2(src, dst, typ, yflat, w):
    kern = pl.kernel(
        _sc2_body,
        out_type=jax.ShapeDtypeStruct((NC, NP, H), jnp.float32),
        mesh=_sc_mesh(),
        scratch_types=[
            pltpu.VMEM((EB,), jnp.int32),
            pltpu.VMEM((EB,), jnp.int32),
            pltpu.VMEM((EB,), jnp.int32),
            pltpu.VMEM((EB,), jnp.float32),
            pltpu.VMEM((CH, H), jnp.float32),
            pltpu.VMEM((CH, H), jnp.float32),
            pltpu.VMEM((CH, H), jnp.float32),
            pltpu.VMEM((CH, H), jnp.float32),
            pltpu.VMEM((CH,), jnp.int32),
            pltpu.VMEM((CH,), jnp.int32),
            pltpu.VMEM((CH,), jnp.int32),
            pltpu.VMEM((CH,), jnp.int32),
            pltpu.VMEM((CH,), jnp.int32),
            pltpu.VMEM((CH,), jnp.int32),
            pltpu.VMEM_SHARED((NP, H), jnp.float32),
            pltpu.SemaphoreType.DMA,
            pltpu.SemaphoreType.DMA,
            pltpu.SemaphoreType.DMA,
            pltpu.SemaphoreType.DMA,
            pltpu.SemaphoreType.DMA,
        ],
        name="rgcn_sc_layer2",
    )
    return kern(src, dst, typ, yflat, w)


def _tc1_body(hp0, hp1, root0, b0, w1t, root1, y_out, z_out):
    h = jnp.maximum(hp0[0] + hp1[0] + root0[...] + b0[...], 0.0)
    y_out[...] = jnp.dot(h, w1t[...], preferred_element_type=jnp.float32)
    z_out[...] = jnp.dot(h, root1[...], preferred_element_type=jnp.float32)


def _run_tc1(hpart, root0, b0, w1t, root1):
    BN = 1000
    grid = (N // BN,)
    return pl.pallas_call(
        _tc1_body,
        grid=grid,
        in_specs=[
            pl.BlockSpec((1, BN, H), lambda i: (0, i, 0)),
            pl.BlockSpec((1, BN, H), lambda i: (1, i, 0)),
            pl.BlockSpec((BN, H), lambda i: (i, 0)),
            pl.BlockSpec((1, H), lambda i: (0, 0)),
            pl.BlockSpec((H, R * C), lambda i: (0, 0)),
            pl.BlockSpec((H, C), lambda i: (0, 0)),
        ],
        out_specs=[
            pl.BlockSpec((BN, R * C), lambda i: (i, 0)),
            pl.BlockSpec((BN, C), lambda i: (i, 0)),
        ],
        out_shape=[
            jax.ShapeDtypeStruct((N, R * C), jnp.float32),
            jax.ShapeDtypeStruct((N, C), jnp.float32),
        ],
    )(hpart, hpart, root0, b0, w1t, root1)


def _tc2_body(o0, o1, z, b1, out):
    ow = o0[0] + o1[0]
    slog = z[...] + b1[...]
    for g in range(H // C):
        slog = slog + ow[:, g * C:(g + 1) * C]
    m = jnp.max(slog, axis=1, keepdims=True)
    ex = jnp.exp(slog - m)
    lse = jnp.log(jnp.sum(ex, axis=1, keepdims=True))
    out[...] = slog - m - lse


def _run_tc2(opart, z, b1):
    BN = 1000
    grid = (N // BN,)
    return pl.pallas_call(
        _tc2_body,
        grid=grid,
        in_specs=[
            pl.BlockSpec((1, BN, H), lambda i: (0, i, 0)),
            pl.BlockSpec((1, BN, H), lambda i: (1, i, 0)),
            pl.BlockSpec((BN, C), lambda i: (i, 0)),
            pl.BlockSpec((1, C), lambda i: (0, 0)),
        ],
        out_specs=pl.BlockSpec((BN, C), lambda i: (i, 0)),
        out_shape=jax.ShapeDtypeStruct((N, C), jnp.float32),
    )(opart, opart, z, b1)


def kernel(edge_index, edge_type, W0, root0, b0, W1, root1, b1):
    src = edge_index[0]
    dst = edge_index[1]
    typ = edge_type

    w0flat = W0.reshape(R * N, H)
    w1t = jnp.transpose(W1, (1, 0, 2)).reshape(H, R * C)

    hpart, w = _run_sc1(src, dst, typ, w0flat)
    y2, z = _run_tc1(hpart, root0, b0.reshape(1, H), w1t, root1)
    yflat = y2.reshape(N * 2, H)
    opart = _run_sc2(src, dst, typ, yflat, w)
    out = _run_tc2(opart, z, b1.reshape(1, C))
    return out


# speedup vs baseline: 1.1939x; 1.0994x over previous
import jax
import jax.numpy as jnp
from jax import lax
from jax.experimental import pallas as pl
from jax.experimental.pallas import tpu as pltpu
from jax.experimental.pallas import tpu_sc as plsc

N = 10000
E = 320000
R = 16
H = 128
C = 16
NR = N * R

NC = 2
NS = 16
CH = 80
EB = 2000

EH = E // NS
ET = E // (NC * NS)
NP1 = 10240
RT1 = NP1 // NS
NP = 10112
RT = NP // NS


def _sc_mesh():
    return plsc.VectorSubcoreMesh(core_axis_name="c", subcore_axis_name="s")


def _idx_l1(eb1, eb2, eb3, b, gidx_v, dstc_v):
    for k in range(5):
        sv = eb1[pl.ds(b + k * 16, 16)]
        dv = eb2[pl.ds(b + k * 16, 16)]
        tv = eb3[pl.ds(b + k * 16, 16)]
        gidx_v[pl.ds(k * 16, 16)] = tv * N + sv
        dstc_v[pl.ds(k * 16, 16)] = dv


def _scale_rows(rows, wbuf, b):
    for k in range(5):
        wv = wbuf[pl.ds(b + k * 16, 16)]
        for jj in range(16):
            ws = wv[jj]
            r = k * 16 + jj
            for f in range(8):
                rows[r, pl.ds(f * 16, 16)] = rows[r, pl.ds(f * 16, 16)] * ws


def _sc1_body(src_hbm, dst_hbm, typ_hbm, w0_hbm,
              hpart_hbm, w_hbm,
              eb1, eb2, eb3, wbuf, fbuf,
              rowsA, rowsB,
              gidxA, gidxB, dstcA, dstcB,
              hseg2d, hseg2dB, ones80,
              cnt_sh, acc_sh,
              gsem0, gsem1, wsem0, ssemA, ssemB, stsem, hsem, hsemB):
    c = lax.axis_index("c")
    s = lax.axis_index("s")

    zero16 = jnp.zeros((16,), jnp.float32)
    one16 = jnp.ones((16,), jnp.float32)

    @pl.loop(0, 1008 // 16)
    def _(i):
        fbuf[pl.ds(i * 16, 16)] = zero16

    @pl.loop(0, CH)
    def _(i):
        for f in range(8):
            rowsA[i, pl.ds(f * 16, 16)] = zero16

    for k in range(5):
        ones80[pl.ds(k * 16, 16)] = one16

    for j in range(NR // NS // 1000):
        pltpu.sync_copy(fbuf.at[pl.ds(0, 1000)],
                        cnt_sh.at[pl.ds(s * (NR // NS) + j * 1000, 1000)])
    for j in range(RT1 // CH):
        pltpu.sync_copy(rowsA, acc_sh.at[pl.ds(s * RT1 + j * CH, CH)])

    plsc.subcore_barrier()

    def _hist_half(blk, hseg, hsem):
        eb = s * EH + blk * EB
        d1 = pltpu.async_copy(dst_hbm.at[pl.ds(eb, EB)], eb1, stsem)
        d2 = pltpu.async_copy(typ_hbm.at[pl.ds(eb, EB)], eb2, stsem)
        d1.wait()
        d2.wait()
        for j in range(EB // CH):
            for k in range(5):
                dv = eb1[pl.ds(j * CH + k * 16, 16)]
                tv = eb2[pl.ds(j * CH + k * 16, 16)]
                hseg[j, pl.ds(k * 16, 16)] = dv * R + tv
            pltpu.async_copy(ones80, cnt_sh.at[hseg.at[j]], hsem, add=True)

    def _hist_drain(hsem):
        for j in range(EB // CH):
            pltpu.make_async_copy(w_hbm.at[pl.ds(0, CH)], ones80, hsem).wait()

    @pl.loop(0, EH // EB // 2)
    def _(g):
        @pl.when(g > 0)
        def _():
            _hist_drain(hsem)

        _hist_half(2 * g, hseg2d, hsem)

        @pl.when(g > 0)
        def _():
            _hist_drain(hsemB)

        _hist_half(2 * g + 1, hseg2dB, hsemB)

    _hist_drain(hsem)
    _hist_drain(hsemB)

    plsc.subcore_barrier()

    @pl.loop(0, ET // EB)
    def _(blk):
        eb = c * (E // NC) + s * ET + blk * EB
        d1 = pltpu.async_copy(src_hbm.at[pl.ds(eb, EB)], eb1, stsem)
        d2 = pltpu.async_copy(dst_hbm.at[pl.ds(eb, EB)], eb2, stsem)
        d3 = pltpu.async_copy(typ_hbm.at[pl.ds(eb, EB)], eb3, stsem)
        d1.wait()
        d2.wait()
        d3.wait()

        wdescs = []
        for j in range(EB // CH):
            for k in range(5):
                dv = eb2[pl.ds(j * CH + k * 16, 16)]
                tv = eb3[pl.ds(j * CH + k * 16, 16)]
                hseg2d[j, pl.ds(k * 16, 16)] = dv * R + tv
            wdescs.append(pltpu.async_copy(
                cnt_sh.at[hseg2d.at[j]],
                wbuf.at[pl.ds(j * CH, CH)], wsem0))
        for d in wdescs:
            d.wait()

        @pl.loop(0, EB // 16)
        def _(i):
            v = wbuf[pl.ds(i * 16, 16)]
            wbuf[pl.ds(i * 16, 16)] = 1.0 / jnp.maximum(v, 1.0)

        @pl.loop(0, EB // CH // 2)
        def _(t):
            b0 = (2 * t) * CH
            b1 = b0 + CH
            _idx_l1(eb1, eb2, eb3, b0, gidxA, dstcA)

            @pl.when(t > 0)
            def _():
                pltpu.make_async_copy(
                    w0_hbm.at[pl.ds(0, CH)], rowsA, ssemA).wait()

            dg0 = pltpu.async_copy(w0_hbm.at[gidxA], rowsA, gsem0)
            _idx_l1(eb1, eb2, eb3, b1, gidxB, dstcB)

            @pl.when(t > 0)
            def _():
                pltpu.make_async_copy(
                    w0_hbm.at[pl.ds(0, CH)], rowsB, ssemB).wait()

            dg1 = pltpu.async_copy(w0_hbm.at[gidxB], rowsB, gsem1)

            dg0.wait()
            _scale_rows(rowsA, wbuf, b0)
            pltpu.async_copy(rowsA, acc_sh.at[dstcA], ssemA, add=True)

            dg1.wait()
            _scale_rows(rowsB, wbuf, b1)
            pltpu.async_copy(rowsB, acc_sh.at[dstcB], ssemB, add=True)

        pltpu.make_async_copy(w0_hbm.at[pl.ds(0, CH)], rowsA, ssemA).wait()
        pltpu.make_async_copy(w0_hbm.at[pl.ds(0, CH)], rowsB, ssemB).wait()

        b = (EB // CH - 1) * CH
        _idx_l1(eb1, eb2, eb3, b, gidxA, dstcA)
        dg0 = pltpu.async_copy(w0_hbm.at[gidxA], rowsA, gsem0)
        dg0.wait()
        _scale_rows(rowsA, wbuf, b)
        pltpu.sync_copy(rowsA, acc_sh.at[dstcA], add=True)

        pltpu.sync_copy(wbuf, w_hbm.at[pl.ds(eb, EB)])

    plsc.subcore_barrier()

    pltpu.sync_copy(acc_sh.at[pl.ds(s * RT1, RT1)],
                    hpart_hbm.at[c, pl.ds(s * RT1, RT1)])


def _run_sc1(src, dst, typ, w0flat):
    kern = pl.kernel(
        _sc1_body,
        out_type=[
            jax.ShapeDtypeStruct((NC, NP1, H), jnp.float32),
            jax.ShapeDtypeStruct((E,), jnp.float32),
        ],
        mesh=_sc_mesh(),
        scratch_types=[
            pltpu.VMEM((EB,), jnp.int32),
            pltpu.VMEM((EB,), jnp.int32),
            pltpu.VMEM((EB,), jnp.int32),
            pltpu.VMEM((EB,), jnp.float32),
            pltpu.VMEM((1008,), jnp.float32),
            pltpu.VMEM((CH, H), jnp.float32),
            pltpu.VMEM((CH, H), jnp.float32),
            pltpu.VMEM((CH,), jnp.int32),
            pltpu.VMEM((CH,), jnp.int32),
            pltpu.VMEM((CH,), jnp.int32),
            pltpu.VMEM((CH,), jnp.int32),
            pltpu.VMEM((EB // CH, CH), jnp.int32),
            pltpu.VMEM((EB // CH, CH), jnp.int32),
            pltpu.VMEM((CH,), jnp.float32),
            pltpu.VMEM_SHARED((NR,), jnp.float32),
            pltpu.VMEM_SHARED((NP1, H), jnp.float32),
            pltpu.SemaphoreType.DMA,
            pltpu.SemaphoreType.DMA,
            pltpu.SemaphoreType.DMA,
            pltpu.SemaphoreType.DMA,
            pltpu.SemaphoreType.DMA,
            pltpu.SemaphoreType.DMA,
            pltpu.SemaphoreType.DMA,
            pltpu.SemaphoreType.DMA,
        ],
        name="rgcn_sc_layer1",
    )
    return kern(src, dst, typ, w0flat)


def _idx_l2(eb1, eb2, eb3, b, gidx_v, dstc_v):
    for k in range(5):
        sv = eb1[pl.ds(b + k * 16, 16)]
        dv = eb2[pl.ds(b + k * 16, 16)]
        tv = eb3[pl.ds(b + k * 16, 16)]
        gidx_v[pl.ds(k * 16, 16)] = sv * 2 + (tv >> 3)
        dstc_v[pl.ds(k * 16, 16)] = dv


def _scale_och(och, rows3, wstage, eb3, b, offv):
    for k in range(5):
        wv = wstage[pl.ds(b + k * 16, 16)]
        tvv = eb3[pl.ds(b + k * 16, 16)]
        offv[pl.ds(k * 16, 16)] = (tvv & 7) * C
        for jj in range(16):
            r = k * 16 + jj
            off = (tvv[jj] & 7) * C
            och[r, pl.ds(off, 16)] = rows3[r, pl.ds(off, 16)] * wv[jj]


def _clear_och(och, offv):
    zero16 = jnp.zeros((16,), jnp.float32)
    for k in range(5):
        ov = offv[pl.ds(k * 16, 16)]
        for jj in range(16):
            r = k * 16 + jj
            och[r, pl.ds(ov[jj], 16)] = zero16


def _sc2_body(src_hbm, dst_hbm, typ_hbm, y_hbm, w_hbm,
              opart_hbm,
              eb1, eb2, eb3, wstage, rows3A, rows3B,
              ochA, ochB, gidxA, gidxB, dstcA, dstcB, offA, offB, oacc_sh,
              gsem0, gsem1, ssemA, ssemB, stsem):
    c = lax.axis_index("c")
    s = lax.axis_index("s")

    zero16 = jnp.zeros((16,), jnp.float32)

    @pl.loop(0, CH)
    def _(i):
        for f in range(H // 16):
            ochA[i, pl.ds(f * 16, 16)] = zero16
            ochB[i, pl.ds(f * 16, 16)] = zero16

    for j in range(RT // CH):
        pltpu.sync_copy(ochA, oacc_sh.at[pl.ds(s * RT + j * CH, CH)])
    pltpu.sync_copy(ochA.at[pl.ds(0, RT % CH)],
                    oacc_sh.at[pl.ds(s * RT + (RT // CH) * CH, RT % CH)])

    plsc.subcore_barrier()

    @pl.loop(0, ET // EB)
    def _(blk):
        eb = c * (E // NC) + s * ET + blk * EB
        d1 = pltpu.async_copy(src_hbm.at[pl.ds(eb, EB)], eb1, stsem)
        d2 = pltpu.async_copy(dst_hbm.at[pl.ds(eb, EB)], eb2, stsem)
        d3 = pltpu.async_copy(typ_hbm.at[pl.ds(eb, EB)], eb3, stsem)
        d4 = pltpu.async_copy(w_hbm.at[pl.ds(eb, EB)], wstage, stsem)
        d1.wait()
        d2.wait()
        d3.wait()
        d4.wait()

        @pl.loop(0, EB // CH // 2)
        def _(t):
            b0 = (2 * t) * CH
            b1 = b0 + CH
            _idx_l2(eb1, eb2, eb3, b0, gidxA, dstcA)
            dg0 = pltpu.async_copy(y_hbm.at[gidxA], rows3A, gsem0)
            _idx_l2(eb1, eb2, eb3, b1, gidxB, dstcB)
            dg1 = pltpu.async_copy(y_hbm.at[gidxB], rows3B, gsem1)

            dg0.wait()

            @pl.when(t > 0)
            def _():
                pltpu.make_async_copy(
                    y_hbm.at[pl.ds(0, CH)], ochA, ssemA).wait()
                _clear_och(ochA, offA)

            _scale_och(ochA, rows3A, wstage, eb3, b0, offA)
            pltpu.async_copy(ochA, oacc_sh.at[dstcA], ssemA, add=True)

            dg1.wait()

            @pl.when(t > 0)
            def _():
                pltpu.make_async_copy(
                    y_hbm.at[pl.ds(0, CH)], ochB, ssemB).wait()
                _clear_och(ochB, offB)

            _scale_och(ochB, rows3B, wstage, eb3, b1, offB)
            pltpu.async_copy(ochB, oacc_sh.at[dstcB], ssemB, add=True)

        pltpu.make_async_copy(y_hbm.at[pl.ds(0, CH)], ochA, ssemA).wait()
        _clear_och(ochA, offA)
        pltpu.make_async_copy(y_hbm.at[pl.ds(0, CH)], ochB, ssemB).wait()
        _clear_och(ochB, offB)

        b = (EB // CH - 1) * CH
        _idx_l2(eb1, eb2, eb3, b, gidxA, dstcA)
        dg0 = pltpu.async_copy(y_hbm.at[gidxA], rows3A, gsem0)
        dg0.wait()
        _scale_och(ochA, rows3A, wstage, eb3, b, offA)
        pltpu.sync_copy(ochA, oacc_sh.at[dstcA], add=True)
        _clear_och(ochA, offA)

    plsc.subcore_barrier()

    pltpu.sync_copy(oacc_sh.at[pl.ds(s * RT, RT)],
                    opart_hbm.at[c, pl.ds(s * RT, RT)])


def _run_sc2(src, dst, typ, yflat, w):
    kern = pl.kernel(
        _sc2_body,
        out_type=jax.ShapeDtypeStruct((NC, NP, H), jnp.float32),
        mesh=_sc_mesh(),
        scratch_types=[
            pltpu.VMEM((EB,), jnp.int32),
            pltpu.VMEM((EB,), jnp.int32),
            pltpu.VMEM((EB,), jnp.int32),
            pltpu.VMEM((EB,), jnp.float32),
            pltpu.VMEM((CH, H), jnp.float32),
            pltpu.VMEM((CH, H), jnp.float32),
            pltpu.VMEM((CH, H), jnp.float32),
            pltpu.VMEM((CH, H), jnp.float32),
            pltpu.VMEM((CH,), jnp.int32),
            pltpu.VMEM((CH,), jnp.int32),
            pltpu.VMEM((CH,), jnp.int32),
            pltpu.VMEM((CH,), jnp.int32),
            pltpu.VMEM((CH,), jnp.int32),
            pltpu.VMEM((CH,), jnp.int32),
            pltpu.VMEM_SHARED((NP, H), jnp.float32),
            pltpu.SemaphoreType.DMA,
            pltpu.SemaphoreType.DMA,
            pltpu.SemaphoreType.DMA,
            pltpu.SemaphoreType.DMA,
            pltpu.SemaphoreType.DMA,
        ],
        name="rgcn_sc_layer2",
    )
    return kern(src, dst, typ, yflat, w)


def _tc1_body(hp0, hp1, root0, b0, w1t, root1, y_out, z_out):
    h = jnp.maximum(hp0[0] + hp1[0] + root0[...] + b0[...], 0.0)
    y_out[...] = jnp.dot(h, w1t[...], preferred_element_type=jnp.float32)
    z_out[...] = jnp.dot(h, root1[...], preferred_element_type=jnp.float32)


def _run_tc1(hpart, root0, b0, w1t, root1):
    BN = 1000
    grid = (N // BN,)
    return pl.pallas_call(
        _tc1_body,
        grid=grid,
        in_specs=[
            pl.BlockSpec((1, BN, H), lambda i: (0, i, 0)),
            pl.BlockSpec((1, BN, H), lambda i: (1, i, 0)),
            pl.BlockSpec((BN, H), lambda i: (i, 0)),
            pl.BlockSpec((1, H), lambda i: (0, 0)),
            pl.BlockSpec((H, R * C), lambda i: (0, 0)),
            pl.BlockSpec((H, C), lambda i: (0, 0)),
        ],
        out_specs=[
            pl.BlockSpec((BN, R * C), lambda i: (i, 0)),
            pl.BlockSpec((BN, C), lambda i: (i, 0)),
        ],
        out_shape=[
            jax.ShapeDtypeStruct((N, R * C), jnp.float32),
            jax.ShapeDtypeStruct((N, C), jnp.float32),
        ],
    )(hpart, hpart, root0, b0, w1t, root1)


def _tc2_body(o0, o1, z, b1, out):
    ow = o0[0] + o1[0]
    slog = z[...] + b1[...]
    for g in range(H // C):
        slog = slog + ow[:, g * C:(g + 1) * C]
    m = jnp.max(slog, axis=1, keepdims=True)
    ex = jnp.exp(slog - m)
    lse = jnp.log(jnp.sum(ex, axis=1, keepdims=True))
    out[...] = slog - m - lse


def _run_tc2(opart, z, b1):
    BN = 1000
    grid = (N // BN,)
    return pl.pallas_call(
        _tc2_body,
        grid=grid,
        in_specs=[
            pl.BlockSpec((1, BN, H), lambda i: (0, i, 0)),
            pl.BlockSpec((1, BN, H), lambda i: (1, i, 0)),
            pl.BlockSpec((BN, C), lambda i: (i, 0)),
            pl.BlockSpec((1, C), lambda i: (0, 0)),
        ],
        out_specs=pl.BlockSpec((BN, C), lambda i: (i, 0)),
        out_shape=jax.ShapeDtypeStruct((N, C), jnp.float32),
    )(opart, opart, z, b1)


def kernel(edge_index, edge_type, W0, root0, b0, W1, root1, b1):
    src = edge_index[0]
    dst = edge_index[1]
    typ = edge_type

    w0flat = W0.reshape(R * N, H)
    w1t = jnp.transpose(W1, (1, 0, 2)).reshape(H, R * C)

    hpart, w = _run_sc1(src, dst, typ, w0flat)
    y2, z = _run_tc1(hpart, root0, b0.reshape(1, H), w1t, root1)
    yflat = y2.reshape(N * 2, H)
    opart = _run_sc2(src, dst, typ, yflat, w)
    out = _run_tc2(opart, z, b1.reshape(1, C))
    return out
